# Initial kernel scaffold; baseline (speedup 1.0000x reference)
import jax
import jax.numpy as jnp
from jax.experimental import pallas as pl


def kernel(token_ids, token_table, pos_table):
    raise NotImplementedError("write your pallas kernel here")


# speedup vs baseline: 1.1564x; 1.1564x over previous
import functools

import jax
import jax.numpy as jnp
from jax import lax
from jax.experimental import pallas as pl
from jax.experimental.pallas import tpu as pltpu
from jax.experimental.pallas import tpu_sc as plsc

_LANES = 16
_NUM_WORKERS = 32
_NUM_CORES = 2


def _emb_body(batch, s_per_w, embed, seq_len,
              ids_hbm, pos_hbm, table_hbm, out_hbm,
              idx_v, pos_v, rows_v, sem):
    wid = lax.axis_index("s") * _NUM_CORES + lax.axis_index("c")
    base_s = wid * s_per_w

    pltpu.sync_copy(ids_hbm.at[wid], idx_v)
    pltpu.sync_copy(pos_hbm.at[pl.ds(base_s, s_per_w)], pos_v)

    groups = embed // _LANES

    for b in range(batch):
        pltpu.async_copy(table_hbm.at[idx_v.at[b]], rows_v, sem).wait()

        def add_row(i, carry):
            for j in range(groups):
                sl = pl.ds(j * _LANES, _LANES)
                plsc.addupdate(rows_v.at[i, sl], pos_v[i, sl])
            return carry

        lax.fori_loop(0, s_per_w, add_row, 0)

        pltpu.sync_copy(rows_v, out_hbm.at[pl.ds(b * seq_len + base_s, s_per_w)])


def kernel(token_ids, token_table, pos_table):
    batch, seq_len = token_ids.shape
    vocab, embed = token_table.shape
    s_per_w = seq_len // _NUM_WORKERS

    ids = (token_ids.astype(jnp.int32)
           .reshape(batch, _NUM_WORKERS, s_per_w)
           .transpose(1, 0, 2))

    grid_kernel = functools.partial(
        pl.kernel,
        mesh=plsc.VectorSubcoreMesh(core_axis_name="c", subcore_axis_name="s"),
        out_type=jax.ShapeDtypeStruct((batch * seq_len, embed), jnp.float32),
        scratch_types=[
            pltpu.VMEM((batch, s_per_w), jnp.int32),
            pltpu.VMEM((s_per_w, embed), jnp.float32),
            pltpu.VMEM((s_per_w, embed), jnp.float32),
            pltpu.SemaphoreType.DMA,
        ],
    )
    body = grid_kernel(functools.partial(_emb_body, batch, s_per_w, embed, seq_len))
    out = body(ids, pos_table, token_table)
    return out.reshape(batch, seq_len, embed)
